# baseline (speedup 1.0000x reference)
import jax
import jax.numpy as jnp
from jax import lax
from jax.experimental import pallas as pl
from jax.experimental.pallas import tpu as pltpu
from jax.experimental.pallas import tpu_sc as plsc

_NC = 2
_NS = 16
_L = 16
_CHUNK = 128


def _rsqrt16(x):
    i = plsc.bitcast(x, jnp.int32)
    y = plsc.bitcast(jnp.int32(0x5F3759DF) - (i >> 1), jnp.float32)
    for _ in range(3):
        y = y * (1.5 - (0.5 * x) * y * y)
    return y


def _build(B, D):
    NW = _NC * _NS
    BPW = B // NW
    NCHUNK = BPW // _CHUNK
    GPC = _CHUNK // _L

    mesh = plsc.VectorSubcoreMesh(
        core_axis_name="c", subcore_axis_name="s",
        num_cores=_NC, num_subcores=_NS)

    def body(ids_hbm, table_hbm, out_hbm, idx_v, rows_v, sems):
        w = lax.axis_index("s") * _NC + lax.axis_index("c")
        pltpu.sync_copy(ids_hbm.at[pl.ds(w * NCHUNK, NCHUNK)], idx_v)
        copies = [
            pltpu.async_copy(table_hbm.at[idx_v.at[j]],
                             rows_v.at[pl.ds(j * _CHUNK, _CHUNK)],
                             sems.at[j])
            for j in range(NCHUNK)
        ]

        lanes = lax.iota(jnp.int32, _L)

        def group_body(g, carry):
            rows = g * _L + lanes
            acc = [jnp.zeros((_L,), jnp.float32) for _ in range(4)]
            for c in range(D):
                cvec = jnp.full((_L,), c, jnp.int32)
                x = plsc.load_gather(rows_v, [rows, cvec])
                acc[c % 4] = acc[c % 4] + x * x
            ss = (acc[0] + acc[1]) + (acc[2] + acc[3])
            s = _rsqrt16(jnp.maximum(ss, 1e-24))
            for c in range(D):
                cvec = jnp.full((_L,), c, jnp.int32)
                x = plsc.load_gather(rows_v, [rows, cvec])
                plsc.store_scatter(rows_v, [rows, cvec], x * s)
            return carry

        for j in range(NCHUNK):
            copies[j].wait()
            lax.fori_loop(j * GPC, (j + 1) * GPC, group_body, 0)

        pltpu.sync_copy(rows_v, out_hbm.at[pl.ds(w * BPW, BPW)])

    return pl.kernel(
        body,
        out_type=jax.ShapeDtypeStruct((B, D), jnp.float32),
        mesh=mesh,
        compiler_params=pltpu.CompilerParams(
            needs_layout_passes=False, use_tc_tiling_on_sc=False),
        scratch_types=[
            pltpu.VMEM((NCHUNK, _CHUNK), jnp.int32),
            pltpu.VMEM((BPW, D), jnp.float32),
            pltpu.SemaphoreType.DMA((NCHUNK,)),
        ],
    )


def kernel(user_ids, table):
    B = user_ids.shape[0]
    D = table.shape[1]
    ids = user_ids.astype(jnp.int32).reshape(B // _CHUNK, _CHUNK)
    return _build(B, D)(ids, table.astype(jnp.float32))
